# baseline (speedup 1.0000x reference)
import jax
import jax.numpy as jnp
from jax import lax
from jax.experimental import pallas as pl
from jax.experimental.pallas import tpu as pltpu
from jax.experimental.pallas import tpu_sc as plsc

B, S, V, P, D = 1024, 200, 50265, 514, 768
PAD_IDX = 1
N = B * S
NW = 32
NCHUNK = 4
BC = B // NCHUNK
NC_TOK = BC * S
KB = 80
PER_W = NC_TOK // NW
NBLK = PER_W // KB
BS_TC = 8


def _sc_gather_body(ids_hbm, tok_hbm, out_hbm,
                    idx0, idx1, buf0, buf1,
                    gsem0, gsem1, wsem0, wsem1):
    nc = 2
    wid = lax.axis_index("s") * nc + lax.axis_index("c")
    wbase = wid * PER_W

    idx = (idx0, idx1)
    buf = (buf0, buf1)
    gsem = (gsem0, gsem1)
    wsem = (wsem0, wsem1)

    pltpu.sync_copy(ids_hbm.at[pl.ds(wbase, KB)], idx0)
    pltpu.async_copy(tok_hbm.at[idx0], buf0, gsem0)

    def pair_body(h, _):
        for sub in (0, 1):
            g = 2 * h + sub
            cur, nxt = sub, 1 - sub

            def wait_prev_write():
                pltpu.make_async_copy(
                    buf[nxt], out_hbm.at[pl.ds(0, KB)], wsem[nxt]).wait()

            if sub == 1:
                wait_prev_write()
            else:
                pl.when(h > 0)(wait_prev_write)

            def launch_next():
                nbase = wbase + (g + 1) * KB
                pltpu.sync_copy(ids_hbm.at[pl.ds(nbase, KB)], idx[nxt])
                pltpu.async_copy(tok_hbm.at[idx[nxt]], buf[nxt], gsem[nxt])

            if sub == 0:
                launch_next()
            else:
                pl.when(h < NBLK // 2 - 1)(launch_next)

            pltpu.make_async_copy(
                tok_hbm.at[idx[cur]], buf[cur], gsem[cur]).wait()
            pltpu.async_copy(
                buf[cur], out_hbm.at[pl.ds(wbase + g * KB, KB)], wsem[cur])
        return 0

    lax.fori_loop(0, NBLK // 2, pair_body, 0)
    pltpu.make_async_copy(
        buf1, out_hbm.at[pl.ds(0, KB)], wsem1).wait()


def _tc_ln_body(g_ref, tt_ref, pos_ref, ent_ref, gam_ref, bet_ref, o_ref):
    x32 = g_ref[...]
    lo = lax.bitcast_convert_type(x32 << jnp.uint32(16), jnp.float32)
    hi = lax.bitcast_convert_type(x32 & jnp.uint32(0xFFFF0000), jnp.float32)
    x = jnp.concatenate([lo, hi], axis=-1) + pos_ref[...][None]
    ttf = tt_ref[...].astype(jnp.float32)[..., None]
    e0 = ent_ref[0, :][None, None, :]
    de = (ent_ref[1, :] - ent_ref[0, :])[None, None, :]
    x = x + e0 + ttf * de
    mean = jnp.mean(x, axis=-1, keepdims=True)
    xc = x - mean
    var = jnp.mean(xc * xc, axis=-1, keepdims=True)
    o_ref[...] = (xc * lax.rsqrt(var + 1e-5) * gam_ref[...][None, None, :]
                  + bet_ref[...][None, None, :])


def _tc_ln_body_aliased(g_ref, tt_ref, pos_ref, ent_ref, gam_ref, bet_ref,
                        buf_ref, o_ref):
    del buf_ref
    _tc_ln_body(g_ref, tt_ref, pos_ref, ent_ref, gam_ref, bet_ref, o_ref)


@jax.jit
def _run(ids32, tt32, tok_table, pos_table, ent_table, ln_gamma, ln_beta):
    mesh = plsc.VectorSubcoreMesh(core_axis_name="c", subcore_axis_name="s")
    gather = pl.kernel(
        _sc_gather_body,
        out_type=jax.ShapeDtypeStruct((NC_TOK, D // 2), jnp.uint32),
        mesh=mesh,
        scratch_types=[
            pltpu.VMEM((KB,), jnp.int32),
            pltpu.VMEM((KB,), jnp.int32),
            pltpu.VMEM((KB, D // 2), jnp.uint32),
            pltpu.VMEM((KB, D // 2), jnp.uint32),
            pltpu.SemaphoreType.DMA,
            pltpu.SemaphoreType.DMA,
            pltpu.SemaphoreType.DMA,
            pltpu.SemaphoreType.DMA,
        ],
    )

    lo16 = lax.bitcast_convert_type(
        tok_table[:, :D // 2].astype(jnp.bfloat16), jnp.uint16)
    hi16 = lax.bitcast_convert_type(
        tok_table[:, D // 2:].astype(jnp.bfloat16), jnp.uint16)
    tok_pk = (lo16.astype(jnp.uint32)
              | (hi16.astype(jnp.uint32) << jnp.uint32(16)))
    pos_slice = lax.slice_in_dim(pos_table, PAD_IDX + 1, PAD_IDX + 1 + S,
                                 axis=0)
    tt_bs = tt32.reshape(B, S)
    grid_c = BC // BS_TC

    def tc_chunk(c, rows_c, buf):
        common_in_specs = [
            pl.BlockSpec((BS_TC, S, D // 2), lambda i: (i, 0, 0)),
            pl.BlockSpec((BS_TC, S), lambda i, c=c: (c * grid_c + i, 0)),
            pl.BlockSpec((S, D), lambda i: (0, 0)),
            pl.BlockSpec((2, D), lambda i: (0, 0)),
            pl.BlockSpec((D,), lambda i: (0,)),
            pl.BlockSpec((D,), lambda i: (0,)),
        ]
        out_spec = pl.BlockSpec((BS_TC, S, D),
                                lambda i, c=c: (c * grid_c + i, 0, 0))
        args = (rows_c.reshape(BC, S, D // 2), tt_bs, pos_slice, ent_table,
                ln_gamma, ln_beta)
        if buf is None:
            return pl.pallas_call(
                _tc_ln_body,
                grid=(grid_c,),
                in_specs=common_in_specs,
                out_specs=out_spec,
                out_shape=jax.ShapeDtypeStruct((B, S, D), jnp.float32),
            )(*args)
        return pl.pallas_call(
            _tc_ln_body_aliased,
            grid=(grid_c,),
            in_specs=common_in_specs + [pl.BlockSpec(memory_space=pl.ANY)],
            out_specs=out_spec,
            out_shape=jax.ShapeDtypeStruct((B, S, D), jnp.float32),
            input_output_aliases={6: 0},
        )(*args, buf)

    out = None
    for c in range(NCHUNK):
        ids_c = lax.slice_in_dim(ids32, c * NC_TOK, (c + 1) * NC_TOK, axis=0)
        rows_c = gather(ids_c, tok_pk)
        out = tc_chunk(c, rows_c, out)
    return out


def kernel(input_ids, token_type_ids, tok_table, pos_table, ent_table,
           ln_gamma, ln_beta):
    ids32 = input_ids.reshape(-1).astype(jnp.int32)
    tt32 = token_type_ids.reshape(-1).astype(jnp.int32)
    return _run(ids32, tt32, tok_table, pos_table, ent_table,
                ln_gamma, ln_beta)


# speedup vs baseline: 1.0301x; 1.0301x over previous
import jax
import jax.numpy as jnp
from jax import lax
from jax.experimental import pallas as pl
from jax.experimental.pallas import tpu as pltpu
from jax.experimental.pallas import tpu_sc as plsc

B, S, V, P, D = 1024, 200, 50265, 514, 768
PAD_IDX = 1
N = B * S
NW = 32
NCHUNK = 4
BC = B // NCHUNK
NC_TOK = BC * S
KB = 80
PER_W = NC_TOK // NW
NBLK = PER_W // KB
BS_TC = 16


def _sc_gather_body(ids_hbm, tok_hbm, out_hbm,
                    idx0, idx1, buf0, buf1,
                    gsem0, gsem1, wsem0, wsem1):
    nc = 2
    wid = lax.axis_index("s") * nc + lax.axis_index("c")
    wbase = wid * PER_W

    idx = (idx0, idx1)
    buf = (buf0, buf1)
    gsem = (gsem0, gsem1)
    wsem = (wsem0, wsem1)

    pltpu.sync_copy(ids_hbm.at[pl.ds(wbase, KB)], idx0)
    pltpu.async_copy(tok_hbm.at[idx0], buf0, gsem0)

    def pair_body(h, _):
        for sub in (0, 1):
            g = 2 * h + sub
            cur, nxt = sub, 1 - sub

            def wait_prev_write():
                pltpu.make_async_copy(
                    buf[nxt], out_hbm.at[pl.ds(0, KB)], wsem[nxt]).wait()

            if sub == 1:
                wait_prev_write()
            else:
                pl.when(h > 0)(wait_prev_write)

            def launch_next():
                nbase = wbase + (g + 1) * KB
                pltpu.sync_copy(ids_hbm.at[pl.ds(nbase, KB)], idx[nxt])
                pltpu.async_copy(tok_hbm.at[idx[nxt]], buf[nxt], gsem[nxt])

            if sub == 0:
                launch_next()
            else:
                pl.when(h < NBLK // 2 - 1)(launch_next)

            pltpu.make_async_copy(
                tok_hbm.at[idx[cur]], buf[cur], gsem[cur]).wait()
            pltpu.async_copy(
                buf[cur], out_hbm.at[pl.ds(wbase + g * KB, KB)], wsem[cur])
        return 0

    lax.fori_loop(0, NBLK // 2, pair_body, 0)
    pltpu.make_async_copy(
        buf1, out_hbm.at[pl.ds(0, KB)], wsem1).wait()


def _tc_ln_body(g_ref, tt_ref, pos_ref, ent_ref, gam_ref, bet_ref, o_ref):
    x32 = g_ref[...]
    lo = lax.bitcast_convert_type(x32 << jnp.uint32(16), jnp.float32)
    hi = lax.bitcast_convert_type(x32 & jnp.uint32(0xFFFF0000), jnp.float32)
    x = jnp.concatenate([lo, hi], axis=-1) + pos_ref[...][None]
    ttf = tt_ref[...].astype(jnp.float32)[..., None]
    e0 = ent_ref[0, :][None, None, :]
    de = (ent_ref[1, :] - ent_ref[0, :])[None, None, :]
    x = x + e0 + ttf * de
    mean = jnp.mean(x, axis=-1, keepdims=True)
    xc = x - mean
    var = jnp.mean(xc * xc, axis=-1, keepdims=True)
    o_ref[...] = (xc * lax.rsqrt(var + 1e-5) * gam_ref[...][None, None, :]
                  + bet_ref[...][None, None, :])


def _tc_ln_body_aliased(g_ref, tt_ref, pos_ref, ent_ref, gam_ref, bet_ref,
                        buf_ref, o_ref):
    del buf_ref
    _tc_ln_body(g_ref, tt_ref, pos_ref, ent_ref, gam_ref, bet_ref, o_ref)


@jax.jit
def _run(ids32, tt32, tok_table, pos_table, ent_table, ln_gamma, ln_beta):
    mesh = plsc.VectorSubcoreMesh(core_axis_name="c", subcore_axis_name="s")
    gather = pl.kernel(
        _sc_gather_body,
        out_type=jax.ShapeDtypeStruct((NC_TOK, D // 2), jnp.uint32),
        mesh=mesh,
        scratch_types=[
            pltpu.VMEM((KB,), jnp.int32),
            pltpu.VMEM((KB,), jnp.int32),
            pltpu.VMEM((KB, D // 2), jnp.uint32),
            pltpu.VMEM((KB, D // 2), jnp.uint32),
            pltpu.SemaphoreType.DMA,
            pltpu.SemaphoreType.DMA,
            pltpu.SemaphoreType.DMA,
            pltpu.SemaphoreType.DMA,
        ],
    )

    lo16 = lax.bitcast_convert_type(
        tok_table[:, :D // 2].astype(jnp.bfloat16), jnp.uint16)
    hi16 = lax.bitcast_convert_type(
        tok_table[:, D // 2:].astype(jnp.bfloat16), jnp.uint16)
    tok_pk = (lo16.astype(jnp.uint32)
              | (hi16.astype(jnp.uint32) << jnp.uint32(16)))
    pos_slice = lax.slice_in_dim(pos_table, PAD_IDX + 1, PAD_IDX + 1 + S,
                                 axis=0)
    tt_bs = tt32.reshape(B, S)
    grid_c = BC // BS_TC

    def tc_chunk(c, rows_c, buf):
        common_in_specs = [
            pl.BlockSpec((BS_TC, S, D // 2), lambda i: (i, 0, 0)),
            pl.BlockSpec((BS_TC, S), lambda i, c=c: (c * grid_c + i, 0)),
            pl.BlockSpec((S, D), lambda i: (0, 0)),
            pl.BlockSpec((2, D), lambda i: (0, 0)),
            pl.BlockSpec((D,), lambda i: (0,)),
            pl.BlockSpec((D,), lambda i: (0,)),
        ]
        out_spec = pl.BlockSpec((BS_TC, S, D),
                                lambda i, c=c: (c * grid_c + i, 0, 0))
        args = (rows_c.reshape(BC, S, D // 2), tt_bs, pos_slice, ent_table,
                ln_gamma, ln_beta)
        if buf is None:
            return pl.pallas_call(
                _tc_ln_body,
                grid=(grid_c,),
                in_specs=common_in_specs,
                out_specs=out_spec,
                out_shape=jax.ShapeDtypeStruct((B, S, D), jnp.float32),
            )(*args)
        return pl.pallas_call(
            _tc_ln_body_aliased,
            grid=(grid_c,),
            in_specs=common_in_specs + [pl.BlockSpec(memory_space=pl.ANY)],
            out_specs=out_spec,
            out_shape=jax.ShapeDtypeStruct((B, S, D), jnp.float32),
            input_output_aliases={6: 0},
        )(*args, buf)

    out = None
    for c in range(NCHUNK):
        ids_c = lax.slice_in_dim(ids32, c * NC_TOK, (c + 1) * NC_TOK, axis=0)
        rows_c = gather(ids_c, tok_pk)
        out = tc_chunk(c, rows_c, out)
    return out


def kernel(input_ids, token_type_ids, tok_table, pos_table, ent_table,
           ln_gamma, ln_beta):
    ids32 = input_ids.reshape(-1).astype(jnp.int32)
    tt32 = token_type_ids.reshape(-1).astype(jnp.int32)
    return _run(ids32, tt32, tok_table, pos_table, ent_table,
                ln_gamma, ln_beta)
